# Initial kernel scaffold; baseline (speedup 1.0000x reference)
import jax
import jax.numpy as jnp
from jax.experimental import pallas as pl


def kernel(input, embed_weight):
    raise NotImplementedError("write your pallas kernel here")


# speedup vs baseline: 1.8775x; 1.8775x over previous
import jax
import jax.numpy as jnp
from jax import lax
from jax.experimental import pallas as pl
from jax.experimental.pallas import tpu as pltpu
from jax.experimental.pallas import tpu_sc as plsc

_NC, _NS, _LANES = 2, 16, 16
_NW = _NC * _NS


def kernel(input, embed_weight):
    L = input.shape[1]
    D = embed_weight.shape[1]
    per_w = L // _NW
    groups = per_w // _LANES
    idx = input.reshape(L).astype(jnp.int32)

    mesh = plsc.VectorSubcoreMesh(core_axis_name="c", subcore_axis_name="s")

    @pl.kernel(
        out_type=jax.ShapeDtypeStruct((L * D,), embed_weight.dtype),
        mesh=mesh,
        compiler_params=pltpu.CompilerParams(needs_layout_passes=False),
        scratch_types=[
            pltpu.VMEM(embed_weight.shape, embed_weight.dtype),
            pltpu.VMEM((per_w,), jnp.int32),
            pltpu.VMEM((per_w * D,), embed_weight.dtype),
        ],
    )
    def _embed_kernel(table_hbm, idx_hbm, out_hbm, table_v, idx_v, out_v):
        wid = lax.axis_index("s") * _NC + lax.axis_index("c")
        pltpu.sync_copy(table_hbm, table_v)
        pltpu.sync_copy(idx_hbm.at[pl.ds(wid * per_w, per_w)], idx_v)

        lanes = lax.iota(jnp.int32, _LANES)
        i_pat = [(lanes + 16 * j) // D for j in range(D)]
        d_pat = [(lanes + 16 * j) % D for j in range(D)]

        @pl.loop(0, groups)
        def _(g):
            for j in range(D):
                rows = plsc.load_gather(idx_v, [i_pat[j] + _LANES * g])
                vals = plsc.load_gather(table_v, [rows, d_pat[j]])
                out_v[pl.ds(_LANES * D * g + _LANES * j, _LANES)] = vals

        pltpu.sync_copy(out_v, out_hbm.at[pl.ds(wid * per_w * D, per_w * D)])

    return _embed_kernel(embed_weight, idx).reshape(1, L, D)
